# Initial kernel scaffold; baseline (speedup 1.0000x reference)
import jax
import jax.numpy as jnp
from jax.experimental import pallas as pl


def kernel(time_idxs, te):
    raise NotImplementedError("write your pallas kernel here")


# speedup vs baseline: 7.8429x; 7.8429x over previous
import functools

import jax
import jax.numpy as jnp
from jax.experimental import pallas as pl

D_EMBED = 64
MAX_LEN = 200
HIST = 200
B_BLK = 64


def _gather_block(idx_ref, te_ref, out_ref):
    idx = idx_ref[...]
    table = te_ref[...]
    idx3 = jnp.broadcast_to(idx[:, :, None], (B_BLK, HIST, MAX_LEN))
    cols = jax.lax.broadcasted_iota(jnp.int32, (B_BLK, HIST, MAX_LEN), 2)
    onehot = (idx3 == cols).astype(jnp.float32).reshape(B_BLK * HIST, MAX_LEN)
    rows = jnp.dot(onehot, table, preferred_element_type=jnp.float32)
    out_ref[...] = rows.reshape(B_BLK, HIST, D_EMBED)


@jax.jit
def kernel(time_idxs, te):
    batch, hist = time_idxs.shape
    grid = (batch // B_BLK,)
    return pl.pallas_call(
        _gather_block,
        grid=grid,
        in_specs=[
            pl.BlockSpec((B_BLK, hist), lambda i: (i, 0)),
            pl.BlockSpec((MAX_LEN, D_EMBED), lambda i: (0, 0)),
        ],
        out_specs=pl.BlockSpec((B_BLK, hist, D_EMBED), lambda i: (i, 0, 0)),
        out_shape=jax.ShapeDtypeStruct((batch, hist, D_EMBED), jnp.float32),
    )(time_idxs, te)


# speedup vs baseline: 8.1027x; 1.0331x over previous
import functools

import jax
import jax.numpy as jnp
from jax.experimental import pallas as pl

D_EMBED = 64
MAX_LEN = 200
HIST = 200
B_BLK = 128


def _gather_block(idx_ref, te_ref, out_ref):
    idx = idx_ref[...]
    table = te_ref[...]
    idx3 = jnp.broadcast_to(idx[:, :, None], (B_BLK, HIST, MAX_LEN))
    cols = jax.lax.broadcasted_iota(jnp.int32, (B_BLK, HIST, MAX_LEN), 2)
    onehot = (idx3 == cols).astype(jnp.float32).reshape(B_BLK * HIST, MAX_LEN)
    rows = jnp.dot(onehot, table, preferred_element_type=jnp.float32)
    out_ref[...] = rows.reshape(B_BLK, HIST, D_EMBED)


@jax.jit
def kernel(time_idxs, te):
    batch, hist = time_idxs.shape
    grid = (batch // B_BLK,)
    return pl.pallas_call(
        _gather_block,
        grid=grid,
        in_specs=[
            pl.BlockSpec((B_BLK, hist), lambda i: (i, 0)),
            pl.BlockSpec((MAX_LEN, D_EMBED), lambda i: (0, 0)),
        ],
        out_specs=pl.BlockSpec((B_BLK, hist, D_EMBED), lambda i: (i, 0, 0)),
        out_shape=jax.ShapeDtypeStruct((batch, hist, D_EMBED), jnp.float32),
    )(time_idxs, te)


# speedup vs baseline: 8.2468x; 1.0178x over previous
import jax
import jax.numpy as jnp
from jax.experimental import pallas as pl
from jax.experimental.pallas import tpu as pltpu

D_EMBED = 64
MAX_LEN = 200
HIST = 200
B_BLK = 64
NSLOTS = 4


def _gather_block(idx_ref, te_ref, out_hbm, scratch, sems):
    i = pl.program_id(0)
    nsteps = pl.num_programs(0)
    slot = jax.lax.rem(i, NSLOTS)

    @pl.when(i >= NSLOTS)
    def _wait_prev():
        pltpu.make_async_copy(
            scratch.at[slot],
            out_hbm.at[pl.ds((i - NSLOTS) * B_BLK, B_BLK)],
            sems.at[slot],
        ).wait()

    idx = idx_ref[...]
    table = te_ref[...]
    idx3 = jnp.broadcast_to(idx[:, :, None], (B_BLK, HIST, MAX_LEN))
    cols = jax.lax.broadcasted_iota(jnp.int32, (B_BLK, HIST, MAX_LEN), 2)
    onehot = (idx3 == cols).astype(jnp.float32).reshape(B_BLK * HIST, MAX_LEN)
    rows = jnp.dot(onehot, table, preferred_element_type=jnp.float32)
    scratch[slot] = rows.reshape(B_BLK, HIST, D_EMBED)

    pltpu.make_async_copy(
        scratch.at[slot],
        out_hbm.at[pl.ds(i * B_BLK, B_BLK)],
        sems.at[slot],
    ).start()

    @pl.when(i == nsteps - 1)
    def _drain():
        for s in range(NSLOTS):
            step = i - (NSLOTS - 1) + s
            pltpu.make_async_copy(
                scratch.at[s],
                out_hbm.at[pl.ds(step * B_BLK, B_BLK)],
                sems.at[s],
            ).wait()


@jax.jit
def kernel(time_idxs, te):
    batch, hist = time_idxs.shape
    grid = (batch // B_BLK,)
    return pl.pallas_call(
        _gather_block,
        grid=grid,
        in_specs=[
            pl.BlockSpec((B_BLK, hist), lambda i: (i, 0)),
            pl.BlockSpec((MAX_LEN, D_EMBED), lambda i: (0, 0)),
        ],
        out_specs=pl.BlockSpec(memory_space=pltpu.HBM),
        out_shape=jax.ShapeDtypeStruct((batch, hist, D_EMBED), jnp.float32),
        scratch_shapes=[
            pltpu.VMEM((NSLOTS, B_BLK, HIST, D_EMBED), jnp.float32),
            pltpu.SemaphoreType.DMA((NSLOTS,)),
        ],
    )(time_idxs, te)
